# baseline (speedup 1.0000x reference)
import functools

import jax
import jax.numpy as jnp
from jax import lax
from jax.experimental import pallas as pl
from jax.experimental.pallas import tpu as pltpu
from jax.experimental.pallas import tpu_sc as plsc

N = 50000
E = 800000
B = 16
NPE = 3125
OBS = 256
NA = 1024
EPE = 512
T = B * EPE

EW = 1000
ER = E // EW
EW1 = 500
CW = 16
NC = 128 // CW

f32 = jnp.float32
i32 = jnp.int32

@functools.cache
def _mesh():
    return plsc.VectorSubcoreMesh(
        core_axis_name="c", subcore_axis_name="s", num_cores=2, num_subcores=16)


_CH = 400


def _tile_chunks(s, body):
    n = N // _CH
    for k in range((n + 15) // 16):
        idx = s + 16 * k

        @pl.when(idx < n)
        def _(idx=idx):
            body(pl.ds(idx * _CH, _CH))


def _edge_stream(tab, acc, srcm, dstm, base, nslabs, sh,
                 idx_s, idx_d, rows, sem):
    def slab(k, carry):
        row = base + sh * k
        pltpu.sync_copy(srcm.at[pl.ds(row, sh)], idx_s)
        pltpu.sync_copy(dstm.at[pl.ds(row, sh)], idx_d)
        for j in range(sh):
            pltpu.async_copy(tab.at[idx_s.at[j]], rows, sem).wait()
            pltpu.sync_copy(rows, acc.at[idx_d.at[j]], add=True)
        return carry

    lax.fori_loop(0, nslabs, slab, 0)


def _sc_agg1_body(x8, srcm, dstm, z8, out_a, out_b,
                  acc, idx_s, idx_d, rows, sem):
    c = lax.axis_index("c")
    s = lax.axis_index("s")
    _tile_chunks(s, lambda sl: pltpu.sync_copy(z8.at[sl], acc.at[sl]))
    plsc.subcore_barrier()
    er = E // EW1
    base = c * (er // 2) + s * (er // 32)
    _edge_stream(x8, acc, srcm, dstm, base, (er // 32) // 2, 2,
                 idx_s, idx_d, rows, sem)
    plsc.subcore_barrier()

    @pl.when(c == 0)
    def _():
        _tile_chunks(s, lambda sl: pltpu.sync_copy(acc.at[sl], out_a.at[sl]))

    @pl.when(c == 1)
    def _():
        _tile_chunks(s, lambda sl: pltpu.sync_copy(acc.at[sl], out_b.at[sl]))


def _sc_agg1(x8, srcm, dstm, z8):
    call = pl.kernel(
        _sc_agg1_body,
        out_type=[jax.ShapeDtypeStruct((N, 8), f32)] * 2,
        mesh=_mesh(),
        scratch_types=[
            pltpu.VMEM_SHARED((N, 8), f32),
            pltpu.VMEM((2, EW1), i32),
            pltpu.VMEM((2, EW1), i32),
            pltpu.VMEM((EW1, 8), f32),
            pltpu.SemaphoreType.DMA,
        ],
        compiler_params=pltpu.CompilerParams(use_tc_tiling_on_sc=False),
    )
    return call(x8, srcm, dstm, z8)


def _sc_agg2_body(*args):
    hs = args[:NC]
    srcm, dstm, z32 = args[NC:NC + 3]
    outs = args[NC + 3:2 * NC + 3]
    acc, idx_s, idx_d, rows, sem = args[2 * NC + 3:]
    c = lax.axis_index("c")
    s = lax.axis_index("s")
    for chunk in range(NC):
        tab, out = hs[chunk], outs[chunk]

        @pl.when(c == chunk // (NC // 2))
        def _(tab=tab, out=out):
            _tile_chunks(s, lambda sl: pltpu.sync_copy(z32.at[sl], acc.at[sl]))
            plsc.subcore_barrier()
            base = s * (ER // 16)
            _edge_stream(tab, acc, srcm, dstm, base, (ER // 16) // 2, 2,
                         idx_s, idx_d, rows, sem)
            plsc.subcore_barrier()
            _tile_chunks(s, lambda sl: pltpu.sync_copy(acc.at[sl], out.at[sl]))
            plsc.subcore_barrier()


def _sc_agg2(hs, srcm, dstm, z32):
    call = pl.kernel(
        _sc_agg2_body,
        out_type=[jax.ShapeDtypeStruct((N, CW), f32)] * NC,
        mesh=_mesh(),
        scratch_types=[
            pltpu.VMEM_SHARED((N, CW), f32),
            pltpu.VMEM((2, EW), i32),
            pltpu.VMEM((2, EW), i32),
            pltpu.VMEM((EW, CW), f32),
            pltpu.SemaphoreType.DMA,
        ],
        compiler_params=pltpu.CompilerParams(use_tc_tiling_on_sc=False),
    )
    return call(*hs, srcm, dstm, z32)


def _sc_gather_body(ne, sg, dg, outs, outd, idxv, rows, sem):
    c = lax.axis_index("c")
    s = lax.axis_index("s")
    wid = s * 2 + c
    for t in range(2):
        r = wid * 2 + t
        pltpu.sync_copy(sg.at[pl.ds(r * 128, 128)], idxv)
        pltpu.async_copy(ne.at[idxv], rows, sem).wait()
        pltpu.sync_copy(rows, outs.at[pl.ds(r * 128, 128)])
        pltpu.sync_copy(dg.at[pl.ds(r * 128, 128)], idxv)
        pltpu.async_copy(ne.at[idxv], rows, sem).wait()
        pltpu.sync_copy(rows, outd.at[pl.ds(r * 128, 128)])


def _sc_gather(ne, sgi, dgi):
    call = pl.kernel(
        _sc_gather_body,
        out_type=[jax.ShapeDtypeStruct((T, 128), f32)] * 2,
        mesh=_mesh(),
        scratch_types=[
            pltpu.VMEM((128,), i32),
            pltpu.VMEM((128, 128), f32),
            pltpu.SemaphoreType.DMA,
        ],
        compiler_params=pltpu.CompilerParams(use_tc_tiling_on_sc=False),
    )
    return call(ne, sgi, dgi)


_RB = 2000


def _dot(a, b):
    return jnp.dot(a, b, preferred_element_type=f32)


def _tc_h_body(x8, aa, ab, ws1, wn1, b1, *outs):
    a = aa[...] + ab[...]
    deg = jnp.maximum(a[:, 5:6], 1.0)
    agg = a / deg
    hv = jnp.maximum(_dot(x8[...], ws1[...]) + _dot(agg, wn1[...]) + b1[...], 0.0)
    for k, o in enumerate(outs):
        o[...] = hv[:, CW * k:CW * k + CW]


def _tc_h(x8, aggA, aggB, ws1p, wn1p, b1r):
    grid = (N // _RB,)
    row8 = pl.BlockSpec((_RB, 8), lambda i: (i, 0))
    wspec = pl.BlockSpec((8, 128), lambda i: (0, 0))
    bspec = pl.BlockSpec((1, 128), lambda i: (0, 0))
    ocw = pl.BlockSpec((_RB, CW), lambda i: (i, 0))
    return pl.pallas_call(
        _tc_h_body,
        grid=grid,
        in_specs=[row8, row8, row8, wspec, wspec, bspec],
        out_specs=[ocw] * NC,
        out_shape=[jax.ShapeDtypeStruct((N, CW), f32)] * NC,
    )(x8, aggA, aggB, ws1p, wn1p, b1r)


def _tc_ne_body(*args):
    hs = args[:NC]
    a2s = args[NC:2 * NC]
    aa, ab = args[2 * NC:2 * NC + 2]
    ws = args[2 * NC + 2:3 * NC + 2]
    vs = args[3 * NC + 2:4 * NC + 2]
    b2 = args[4 * NC + 2]
    out, pm, px = args[4 * NC + 3:]
    i = pl.program_id(0)
    a = aa[...] + ab[...]
    deg = jnp.maximum(a[:, 5:6], 1.0)
    acc = sum(_dot(hs[k][...], ws[k][...]) for k in range(NC))
    acc += sum(_dot(a2s[k][...] / deg, vs[k][...]) for k in range(NC))
    ne = acc + b2[...]
    out[...] = ne

    @pl.when(i == 0)
    def _():
        pm[...] = jnp.zeros((B, 128), f32)
        px[...] = jnp.full((B, 128), -1e30, f32)

    e0 = (_RB * i) // NPE
    row_g = lax.broadcasted_iota(i32, (_RB, 1), 0) + _RB * i
    env = e0 + jnp.where(row_g >= (e0 + 1) * NPE, 1, 0)
    oh = (env == lax.broadcasted_iota(i32, (1, B), 1)).astype(f32)
    pm[...] += lax.dot_general(oh, ne, (((0,), (0,)), ((), ())),
                               preferred_element_type=f32)
    rowsel = lax.broadcasted_iota(i32, (B, 1), 0)
    for k in range(2):
        mx = jnp.max(jnp.where(env == e0 + k, ne, -1e30), axis=0,
                     keepdims=True)
        px[...] = jnp.maximum(
            px[...], jnp.where(rowsel == e0 + k,
                               jnp.broadcast_to(mx, (B, 128)), -1e30))


def _tc_ne(hs, a2s, aggA, aggB, ws2s, wn2s, b2r):
    grid = (N // _RB,)
    rcw = pl.BlockSpec((_RB, CW), lambda i: (i, 0))
    row8 = pl.BlockSpec((_RB, 8), lambda i: (i, 0))
    wspec = pl.BlockSpec((CW, 128), lambda i: (0, 0))
    bspec = pl.BlockSpec((1, 128), lambda i: (0, 0))
    full = pl.BlockSpec((B, 128), lambda i: (0, 0))
    return pl.pallas_call(
        _tc_ne_body,
        grid=grid,
        in_specs=[rcw] * NC + [rcw] * NC + [row8, row8]
        + [wspec] * (2 * NC) + [bspec],
        out_specs=[pl.BlockSpec((_RB, 128), lambda i: (i, 0)), full, full],
        out_shape=[jax.ShapeDtypeStruct((N, 128), f32),
                   jax.ShapeDtypeStruct((B, 128), f32),
                   jax.ShapeDtypeStruct((B, 128), f32)],
    )(*hs, *a2s, aggA, aggB, *ws2s, *wn2s, b2r)


def _ln(x, g, b):
    m = jnp.mean(x, axis=-1, keepdims=True)
    v = jnp.mean((x - m) ** 2, axis=-1, keepdims=True)
    return (x - m) / jnp.sqrt(v + 1e-5) * g + b


def _tc_heads_body(obs, mp, xp,
                   ow1, ob1, og1, obt1, ow2, ob2, og2, obt2,
                   fw0, fw1, fw2, fb, fg, fbt,
                   rw1, rb1, rg1, rbt1, rw2, rb2, rg2, rbt2,
                   cw1, cb1, cw2r, cb2b, pw1, pb1, pw2r, pb2b,
                   ew1c, eb1r,
                   smalls, ctxp):
    o = jnp.maximum(_ln(_dot(obs[...], ow1[...]) + ob1[...], og1[...], obt1[...]), 0.0)
    oe = jnp.maximum(_ln(_dot(o, ow2[...]) + ob2[...], og2[...], obt2[...]), 0.0)
    gin = _dot(oe, fw0[...]) + _dot(mp[...], fw1[...]) + _dot(xp[...], fw2[...])
    g = jnp.maximum(_ln(gin + fb[...], fg[...], fbt[...]), 0.0)
    r = jnp.maximum(_ln(_dot(g, rw1[...]) + rb1[...], rg1[...], rbt1[...]), 0.0)
    r = _ln(_dot(r, rw2[...]) + rb2[...], rg2[...], rbt2[...])
    ctx = jnp.maximum(r + g, 0.0)
    tc = jnp.maximum(_dot(ctx, cw1[...]) + cb1[...], 0.0)
    vals = jnp.sum(tc * cw2r[...], axis=1, keepdims=True) + cb2b[...]
    tp = jnp.maximum(_dot(ctx, pw1[...]) + pb1[...], 0.0)
    pas = jnp.sum(tp * pw2r[...], axis=1, keepdims=True) + pb2b[...]
    col = lax.broadcasted_iota(i32, (B, 128), 1)
    smalls[...] = jnp.where(col == 0, vals, jnp.where(col == 1, pas, 0.0))
    ctxp[...] = _dot(ctx, ew1c[...]) + eb1r[...]


def _tc_heads(obs, mp, xp, args):
    in_shapes = [obs, mp, xp] + list(args)
    return pl.pallas_call(
        _tc_heads_body,
        out_shape=[jax.ShapeDtypeStruct((B, 128), f32)] * 2,
    )(*in_shapes)


def _tc_score_body(gs, gd, ctxp, smalls, ew1a, ew1b, ew2r, eb2b, out):
    t = jnp.maximum(
        _dot(gs[...], ew1a[...]) + _dot(gd[...], ew1b[...]) + ctxp[0], 0.0)
    srow = lax.dot_general(ew2r[...], t, (((1,), (1,)), ((), ()))) + eb2b[0, 0]
    col = lax.broadcasted_iota(i32, (1, NA), 1)
    pad = jnp.concatenate([srow, jnp.full((1, NA - EPE), -100000000.0, f32)], axis=1)
    out[...] = jnp.where(col == EPE, smalls[0, 0, 1], pad)[None]


def _tc_score(gs, gd, ctxp3, smalls3, ew1a, ew1b, ew2r, eb2b):
    erow = pl.BlockSpec((EPE, 128), lambda e: (e, 0))
    row = pl.BlockSpec((1, 1, 128), lambda e: (e, 0, 0))
    wspec = pl.BlockSpec((128, 128), lambda e: (0, 0))
    rspec = pl.BlockSpec((1, 128), lambda e: (0, 0))
    return pl.pallas_call(
        _tc_score_body,
        grid=(B,),
        in_specs=[erow, erow, row, row, wspec, wspec, rspec, rspec],
        out_specs=pl.BlockSpec((1, 1, NA), lambda e: (e, 0, 0)),
        out_shape=jax.ShapeDtypeStruct((B, 1, NA), f32),
    )(gs, gd, ctxp3, smalls3, ew1a, ew1b, ew2r, eb2b)


def kernel(obs, x, edge_index, batch, ptr, swap_src, swap_dst, swap_ptr, params):
    p = params
    src = edge_index[0]
    dst = edge_index[1]
    srcm = src.reshape(ER, EW)
    dstm = dst.reshape(ER, EW)
    x8 = jnp.concatenate(
        [x, jnp.ones((N, 1), f32), jnp.zeros((N, 2), f32)], axis=1)
    z8 = jnp.zeros((N, 8), f32)
    zcw = jnp.zeros((N, CW), f32)

    aggA, aggB = _sc_agg1(x8, src.reshape(E // EW1, EW1),
                          dst.reshape(E // EW1, EW1), z8)
    zpad = jnp.zeros((3, 128), f32)
    ws1p = jnp.concatenate([p["Ws1"], zpad], axis=0)
    wn1p = jnp.concatenate([p["Wn1"], zpad], axis=0)
    hs = _tc_h(x8, aggA, aggB, ws1p, wn1p, p["b1"].reshape(1, 128))

    a2s = _sc_agg2(hs, srcm, dstm, zcw)
    ws2s = [p["Ws2"][CW * c:CW * c + CW] for c in range(NC)]
    wn2s = [p["Wn2"][CW * c:CW * c + CW] for c in range(NC)]
    ne, pm, xp = _tc_ne(hs, a2s, aggA, aggB, ws2s, wn2s,
                        p["b2"].reshape(1, 128))
    mp = pm * (1.0 / NPE)

    def row(v):
        return v.reshape(1, -1)

    heads_args = [
        p["oW1"], row(p["ob1"]), row(p["og1"]), row(p["obt1"]),
        p["oW2"], row(p["ob2"]), row(p["og2"]), row(p["obt2"]),
        p["fW"][0:128], p["fW"][128:256], p["fW"][256:384],
        row(p["fb"]), row(p["fg"]), row(p["fbt"]),
        p["rW1"], row(p["rb1"]), row(p["rg1"]), row(p["rbt1"]),
        p["rW2"], row(p["rb2"]), row(p["rg2"]), row(p["rbt2"]),
        p["cW1"], row(p["cb1"]), p["cW2"].reshape(1, 128),
        jnp.broadcast_to(p["cb2"].reshape(1, 1), (1, 128)),
        p["pW1"], row(p["pb1"]), p["pW2"].reshape(1, 64),
        jnp.broadcast_to(p["pb2"].reshape(1, 1), (1, 128)),
        p["eW1"][256:512], row(p["eb1"]),
    ]
    smalls, ctxp = _tc_heads(obs, mp, xp, heads_args)

    env = jnp.repeat(jnp.arange(B, dtype=i32), EPE)
    off = ptr[:-1][env]
    npg = (ptr[1:] - ptr[:-1])[env]
    sgi = jnp.minimum(swap_src, npg - 1) + off
    dgi = jnp.minimum(swap_dst, npg - 1) + off
    gs, gd = _sc_gather(ne, sgi, dgi)

    logits3 = _tc_score(
        gs, gd, ctxp.reshape(B, 1, 128), smalls.reshape(B, 1, 128),
        p["eW1"][0:128], p["eW1"][128:256],
        p["eW2"].reshape(1, 128),
        jnp.broadcast_to(p["eb2"].reshape(1, 1), (1, 128)))
    logits = logits3.reshape(B, NA)
    values = smalls[:, 0]
    return logits, values


# speedup vs baseline: 1.2126x; 1.2126x over previous
import functools

import jax
import jax.numpy as jnp
from jax import lax
from jax.experimental import pallas as pl
from jax.experimental.pallas import tpu as pltpu
from jax.experimental.pallas import tpu_sc as plsc

N = 50000
E = 800000
B = 16
NPE = 3125
OBS = 256
NA = 1024
EPE = 512
T = B * EPE

EW = 250
ER = E // EW
EW1 = 500

f32 = jnp.float32
i32 = jnp.int32

@functools.cache
def _mesh():
    return plsc.VectorSubcoreMesh(
        core_axis_name="c", subcore_axis_name="s", num_cores=2, num_subcores=16)


_CH = 400


def _tile_chunks(s, body):
    n = N // _CH
    for k in range((n + 15) // 16):
        idx = s + 16 * k

        @pl.when(idx < n)
        def _(idx=idx):
            body(pl.ds(idx * _CH, _CH))


def _edge_stream(tab, acc, srcm, dstm, base, nslabs, sh,
                 idx_s, idx_d, rows, sem):
    def slab(k, carry):
        row = base + sh * k
        pltpu.sync_copy(srcm.at[pl.ds(row, sh)], idx_s)
        pltpu.sync_copy(dstm.at[pl.ds(row, sh)], idx_d)
        for j in range(sh):
            pltpu.async_copy(tab.at[idx_s.at[j]], rows, sem).wait()
            pltpu.sync_copy(rows, acc.at[idx_d.at[j]], add=True)
        return carry

    lax.fori_loop(0, nslabs, slab, 0)


def _sc_agg1_body(x8, srcm, dstm, z8, out_a, out_b,
                  acc, idx_s, idx_d, rows, sem):
    c = lax.axis_index("c")
    s = lax.axis_index("s")
    _tile_chunks(s, lambda sl: pltpu.sync_copy(z8.at[sl], acc.at[sl]))
    plsc.subcore_barrier()
    er = E // EW1
    base = c * (er // 2) + s * (er // 32)
    _edge_stream(x8, acc, srcm, dstm, base, (er // 32) // 2, 2,
                 idx_s, idx_d, rows, sem)
    plsc.subcore_barrier()

    @pl.when(c == 0)
    def _():
        _tile_chunks(s, lambda sl: pltpu.sync_copy(acc.at[sl], out_a.at[sl]))

    @pl.when(c == 1)
    def _():
        _tile_chunks(s, lambda sl: pltpu.sync_copy(acc.at[sl], out_b.at[sl]))


def _sc_agg1(x8, srcm, dstm, z8):
    call = pl.kernel(
        _sc_agg1_body,
        out_type=[jax.ShapeDtypeStruct((N, 8), f32)] * 2,
        mesh=_mesh(),
        scratch_types=[
            pltpu.VMEM_SHARED((N, 8), f32),
            pltpu.VMEM((2, EW1), i32),
            pltpu.VMEM((2, EW1), i32),
            pltpu.VMEM((EW1, 8), f32),
            pltpu.SemaphoreType.DMA,
        ],
        compiler_params=pltpu.CompilerParams(use_tc_tiling_on_sc=False),
    )
    return call(x8, srcm, dstm, z8)


def _sc_agg2_body(h0, h1, h2, h3, srcm, dstm, z32,
                  o0, o1, o2, o3, acc, idx_s, idx_d, rows, sem):
    c = lax.axis_index("c")
    s = lax.axis_index("s")
    tabs = ((h0, o0), (h1, o1), (h2, o2), (h3, o3))
    for chunk in range(4):
        tab, out = tabs[chunk]

        @pl.when(c == chunk // 2)
        def _(tab=tab, out=out):
            _tile_chunks(s, lambda sl: pltpu.sync_copy(z32.at[sl], acc.at[sl]))
            plsc.subcore_barrier()
            base = s * (ER // 16)
            _edge_stream(tab, acc, srcm, dstm, base, (ER // 16) // 8, 8,
                         idx_s, idx_d, rows, sem)
            plsc.subcore_barrier()
            _tile_chunks(s, lambda sl: pltpu.sync_copy(acc.at[sl], out.at[sl]))
            plsc.subcore_barrier()


def _sc_agg2(h0, h1, h2, h3, srcm, dstm, z32):
    call = pl.kernel(
        _sc_agg2_body,
        out_type=[jax.ShapeDtypeStruct((N, 32), f32)] * 4,
        mesh=_mesh(),
        scratch_types=[
            pltpu.VMEM_SHARED((N, 32), f32),
            pltpu.VMEM((8, EW), i32),
            pltpu.VMEM((8, EW), i32),
            pltpu.VMEM((EW, 32), f32),
            pltpu.SemaphoreType.DMA,
        ],
        compiler_params=pltpu.CompilerParams(use_tc_tiling_on_sc=False),
    )
    return call(h0, h1, h2, h3, srcm, dstm, z32)


def _sc_gather_body(ne, sg, dg, outs, outd, idxv, rows, sem):
    c = lax.axis_index("c")
    s = lax.axis_index("s")
    wid = s * 2 + c
    for t in range(2):
        r = wid * 2 + t
        pltpu.sync_copy(sg.at[pl.ds(r * 128, 128)], idxv)
        pltpu.async_copy(ne.at[idxv], rows, sem).wait()
        pltpu.sync_copy(rows, outs.at[pl.ds(r * 128, 128)])
        pltpu.sync_copy(dg.at[pl.ds(r * 128, 128)], idxv)
        pltpu.async_copy(ne.at[idxv], rows, sem).wait()
        pltpu.sync_copy(rows, outd.at[pl.ds(r * 128, 128)])


def _sc_gather(ne, sgi, dgi):
    call = pl.kernel(
        _sc_gather_body,
        out_type=[jax.ShapeDtypeStruct((T, 128), f32)] * 2,
        mesh=_mesh(),
        scratch_types=[
            pltpu.VMEM((128,), i32),
            pltpu.VMEM((128, 128), f32),
            pltpu.SemaphoreType.DMA,
        ],
        compiler_params=pltpu.CompilerParams(use_tc_tiling_on_sc=False),
    )
    return call(ne, sgi, dgi)


_RB = 2000


def _dot(a, b):
    return jnp.dot(a, b, preferred_element_type=f32)


def _tc_h_body(x8, aa, ab, ws1, wn1, b1, o0, o1, o2, o3):
    a = aa[...] + ab[...]
    deg = jnp.maximum(a[:, 5:6], 1.0)
    agg = a / deg
    hv = jnp.maximum(_dot(x8[...], ws1[...]) + _dot(agg, wn1[...]) + b1[...], 0.0)
    o0[...] = hv[:, 0:32]
    o1[...] = hv[:, 32:64]
    o2[...] = hv[:, 64:96]
    o3[...] = hv[:, 96:128]


def _tc_h(x8, aggA, aggB, ws1p, wn1p, b1r):
    grid = (N // _RB,)
    row8 = pl.BlockSpec((_RB, 8), lambda i: (i, 0))
    wspec = pl.BlockSpec((8, 128), lambda i: (0, 0))
    bspec = pl.BlockSpec((1, 128), lambda i: (0, 0))
    o32 = pl.BlockSpec((_RB, 32), lambda i: (i, 0))
    return pl.pallas_call(
        _tc_h_body,
        grid=grid,
        in_specs=[row8, row8, row8, wspec, wspec, bspec],
        out_specs=[o32] * 4,
        out_shape=[jax.ShapeDtypeStruct((N, 32), f32)] * 4,
    )(x8, aggA, aggB, ws1p, wn1p, b1r)


def _tc_ne_body(h0, h1, h2, h3, a0, a1, a2, a3, aa, ab,
                w0, w1, w2, w3, v0, v1, v2, v3, b2, out, pm, px):
    i = pl.program_id(0)
    a = aa[...] + ab[...]
    deg = jnp.maximum(a[:, 5:6], 1.0)
    acc = _dot(h0[...], w0[...]) + _dot(h1[...], w1[...]) \
        + _dot(h2[...], w2[...]) + _dot(h3[...], w3[...])
    acc += _dot(a0[...] / deg, v0[...]) + _dot(a1[...] / deg, v1[...]) \
        + _dot(a2[...] / deg, v2[...]) + _dot(a3[...] / deg, v3[...])
    ne = acc + b2[...]
    out[...] = ne

    @pl.when(i == 0)
    def _():
        pm[...] = jnp.zeros((B, 128), f32)
        px[...] = jnp.full((B, 128), -1e30, f32)

    e0 = (_RB * i) // NPE
    row_g = lax.broadcasted_iota(i32, (_RB, 1), 0) + _RB * i
    env = e0 + jnp.where(row_g >= (e0 + 1) * NPE, 1, 0)
    oh = (env == lax.broadcasted_iota(i32, (1, B), 1)).astype(f32)
    pm[...] += lax.dot_general(oh, ne, (((0,), (0,)), ((), ())),
                               preferred_element_type=f32)
    rowsel = lax.broadcasted_iota(i32, (B, 1), 0)
    for k in range(2):
        mx = jnp.max(jnp.where(env == e0 + k, ne, -1e30), axis=0,
                     keepdims=True)
        px[...] = jnp.maximum(
            px[...], jnp.where(rowsel == e0 + k,
                               jnp.broadcast_to(mx, (B, 128)), -1e30))


def _tc_ne(hs, a2s, aggA, aggB, ws2s, wn2s, b2r):
    grid = (N // _RB,)
    r32 = pl.BlockSpec((_RB, 32), lambda i: (i, 0))
    row8 = pl.BlockSpec((_RB, 8), lambda i: (i, 0))
    wspec = pl.BlockSpec((32, 128), lambda i: (0, 0))
    bspec = pl.BlockSpec((1, 128), lambda i: (0, 0))
    full = pl.BlockSpec((B, 128), lambda i: (0, 0))
    return pl.pallas_call(
        _tc_ne_body,
        grid=grid,
        in_specs=[r32] * 4 + [r32] * 4 + [row8, row8] + [wspec] * 8 + [bspec],
        out_specs=[pl.BlockSpec((_RB, 128), lambda i: (i, 0)), full, full],
        out_shape=[jax.ShapeDtypeStruct((N, 128), f32),
                   jax.ShapeDtypeStruct((B, 128), f32),
                   jax.ShapeDtypeStruct((B, 128), f32)],
    )(*hs, *a2s, aggA, aggB, *ws2s, *wn2s, b2r)


def _ln(x, g, b):
    m = jnp.mean(x, axis=-1, keepdims=True)
    v = jnp.mean((x - m) ** 2, axis=-1, keepdims=True)
    return (x - m) / jnp.sqrt(v + 1e-5) * g + b


def _tc_heads_body(obs, mp, xp,
                   ow1, ob1, og1, obt1, ow2, ob2, og2, obt2,
                   fw0, fw1, fw2, fb, fg, fbt,
                   rw1, rb1, rg1, rbt1, rw2, rb2, rg2, rbt2,
                   cw1, cb1, cw2r, cb2b, pw1, pb1, pw2r, pb2b,
                   ew1c, eb1r,
                   smalls, ctxp):
    o = jnp.maximum(_ln(_dot(obs[...], ow1[...]) + ob1[...], og1[...], obt1[...]), 0.0)
    oe = jnp.maximum(_ln(_dot(o, ow2[...]) + ob2[...], og2[...], obt2[...]), 0.0)
    gin = _dot(oe, fw0[...]) + _dot(mp[...], fw1[...]) + _dot(xp[...], fw2[...])
    g = jnp.maximum(_ln(gin + fb[...], fg[...], fbt[...]), 0.0)
    r = jnp.maximum(_ln(_dot(g, rw1[...]) + rb1[...], rg1[...], rbt1[...]), 0.0)
    r = _ln(_dot(r, rw2[...]) + rb2[...], rg2[...], rbt2[...])
    ctx = jnp.maximum(r + g, 0.0)
    tc = jnp.maximum(_dot(ctx, cw1[...]) + cb1[...], 0.0)
    vals = jnp.sum(tc * cw2r[...], axis=1, keepdims=True) + cb2b[...]
    tp = jnp.maximum(_dot(ctx, pw1[...]) + pb1[...], 0.0)
    pas = jnp.sum(tp * pw2r[...], axis=1, keepdims=True) + pb2b[...]
    col = lax.broadcasted_iota(i32, (B, 128), 1)
    smalls[...] = jnp.where(col == 0, vals, jnp.where(col == 1, pas, 0.0))
    ctxp[...] = _dot(ctx, ew1c[...]) + eb1r[...]


def _tc_heads(obs, mp, xp, args):
    in_shapes = [obs, mp, xp] + list(args)
    return pl.pallas_call(
        _tc_heads_body,
        out_shape=[jax.ShapeDtypeStruct((B, 128), f32)] * 2,
    )(*in_shapes)


def _tc_score_body(gs, gd, ctxp, smalls, ew1a, ew1b, ew2r, eb2b, out):
    t = jnp.maximum(
        _dot(gs[...], ew1a[...]) + _dot(gd[...], ew1b[...]) + ctxp[0], 0.0)
    srow = lax.dot_general(ew2r[...], t, (((1,), (1,)), ((), ()))) + eb2b[0, 0]
    col = lax.broadcasted_iota(i32, (1, NA), 1)
    pad = jnp.concatenate([srow, jnp.full((1, NA - EPE), -100000000.0, f32)], axis=1)
    out[...] = jnp.where(col == EPE, smalls[0, 0, 1], pad)[None]


def _tc_score(gs, gd, ctxp3, smalls3, ew1a, ew1b, ew2r, eb2b):
    erow = pl.BlockSpec((EPE, 128), lambda e: (e, 0))
    row = pl.BlockSpec((1, 1, 128), lambda e: (e, 0, 0))
    wspec = pl.BlockSpec((128, 128), lambda e: (0, 0))
    rspec = pl.BlockSpec((1, 128), lambda e: (0, 0))
    return pl.pallas_call(
        _tc_score_body,
        grid=(B,),
        in_specs=[erow, erow, row, row, wspec, wspec, rspec, rspec],
        out_specs=pl.BlockSpec((1, 1, NA), lambda e: (e, 0, 0)),
        out_shape=jax.ShapeDtypeStruct((B, 1, NA), f32),
    )(gs, gd, ctxp3, smalls3, ew1a, ew1b, ew2r, eb2b)


def kernel(obs, x, edge_index, batch, ptr, swap_src, swap_dst, swap_ptr, params):
    p = params
    src = edge_index[0]
    dst = edge_index[1]
    srcm = src.reshape(ER, EW)
    dstm = dst.reshape(ER, EW)
    x8 = jnp.concatenate(
        [x, jnp.ones((N, 1), f32), jnp.zeros((N, 2), f32)], axis=1)
    z8 = jnp.zeros((N, 8), f32)
    z32 = jnp.zeros((N, 32), f32)

    aggA, aggB = _sc_agg1(x8, src.reshape(E // EW1, EW1),
                          dst.reshape(E // EW1, EW1), z8)
    zpad = jnp.zeros((3, 128), f32)
    ws1p = jnp.concatenate([p["Ws1"], zpad], axis=0)
    wn1p = jnp.concatenate([p["Wn1"], zpad], axis=0)
    hs = _tc_h(x8, aggA, aggB, ws1p, wn1p, p["b1"].reshape(1, 128))

    a2s = _sc_agg2(*hs, srcm, dstm, z32)
    ws2s = [p["Ws2"][32 * c:32 * c + 32] for c in range(4)]
    wn2s = [p["Wn2"][32 * c:32 * c + 32] for c in range(4)]
    ne, pm, xp = _tc_ne(hs, a2s, aggA, aggB, ws2s, wn2s,
                        p["b2"].reshape(1, 128))
    mp = pm * (1.0 / NPE)

    def row(v):
        return v.reshape(1, -1)

    heads_args = [
        p["oW1"], row(p["ob1"]), row(p["og1"]), row(p["obt1"]),
        p["oW2"], row(p["ob2"]), row(p["og2"]), row(p["obt2"]),
        p["fW"][0:128], p["fW"][128:256], p["fW"][256:384],
        row(p["fb"]), row(p["fg"]), row(p["fbt"]),
        p["rW1"], row(p["rb1"]), row(p["rg1"]), row(p["rbt1"]),
        p["rW2"], row(p["rb2"]), row(p["rg2"]), row(p["rbt2"]),
        p["cW1"], row(p["cb1"]), p["cW2"].reshape(1, 128),
        jnp.broadcast_to(p["cb2"].reshape(1, 1), (1, 128)),
        p["pW1"], row(p["pb1"]), p["pW2"].reshape(1, 64),
        jnp.broadcast_to(p["pb2"].reshape(1, 1), (1, 128)),
        p["eW1"][256:512], row(p["eb1"]),
    ]
    smalls, ctxp = _tc_heads(obs, mp, xp, heads_args)

    env = jnp.repeat(jnp.arange(B, dtype=i32), EPE)
    off = ptr[:-1][env]
    npg = (ptr[1:] - ptr[:-1])[env]
    sgi = jnp.minimum(swap_src, npg - 1) + off
    dgi = jnp.minimum(swap_dst, npg - 1) + off
    gs, gd = _sc_gather(ne, sgi, dgi)

    logits3 = _tc_score(
        gs, gd, ctxp.reshape(B, 1, 128), smalls.reshape(B, 1, 128),
        p["eW1"][0:128], p["eW1"][128:256],
        p["eW2"].reshape(1, 128),
        jnp.broadcast_to(p["eb2"].reshape(1, 1), (1, 128)))
    logits = logits3.reshape(B, NA)
    values = smalls[:, 0]
    return logits, values
